# Initial kernel scaffold; baseline (speedup 1.0000x reference)
import jax
import jax.numpy as jnp
from jax.experimental import pallas as pl


def kernel(x, table):
    raise NotImplementedError("write your pallas kernel here")


# speedup vs baseline: 1.8224x; 1.8224x over previous
import functools
import math

import jax
import jax.numpy as jnp
from jax import lax
from jax.experimental import pallas as pl
from jax.experimental.pallas import tpu as pltpu
from jax.experimental.pallas import tpu_sc as plsc

D_MODEL = 128
SCALE = math.sqrt(float(D_MODEL))

NC, NS, L = 2, 16, 16
NW = NC * NS
C = 128
NBUF = 4


def _body(idx_hbm, table_hbm, out_hbm, idx_v, rows_v, *sems):
    gsem = sems[:NBUF]
    ssem = sems[NBUF:]
    n_chunks = idx_v.shape[0]
    n_super = n_chunks // NBUF

    wid = lax.axis_index("s") * NC + lax.axis_index("c")
    base = wid * n_chunks

    pltpu.sync_copy(idx_hbm.at[wid], idx_v)

    def start_gather(b, j):
        pltpu.async_copy(table_hbm.at[idx_v.at[j]], rows_v.at[b], gsem[b])

    def wait_gather(b):
        pltpu.make_async_copy(table_hbm.at[idx_v.at[0]], rows_v.at[b],
                              gsem[b]).wait()

    def start_scatter(b, j):
        pltpu.async_copy(rows_v.at[b],
                         out_hbm.at[pl.ds((base + j) * C, C)], ssem[b])

    def wait_scatter(b):
        pltpu.make_async_copy(rows_v.at[b], out_hbm.at[pl.ds(0, C)],
                              ssem[b]).wait()

    def scale_slot(b):
        def row(r, carry):
            for c8 in range(D_MODEL // L):
                sl = pl.ds(c8 * L, L)
                rows_v[b, r, sl] = rows_v[b, r, sl] * SCALE
            return carry
        lax.fori_loop(0, C, row, None, unroll=4)

    for b in range(NBUF):
        start_gather(b, b)

    def super_body(k, carry):
        j0 = k * NBUF
        for b in range(NBUF):
            wait_gather(b)
            scale_slot(b)
            start_scatter(b, j0 + b)
        for b in range(NBUF):
            wait_scatter(b)
            start_gather(b, j0 + NBUF + b)
        return carry

    lax.fori_loop(0, n_super - 1, super_body, None)

    j0 = (n_super - 1) * NBUF
    for b in range(NBUF):
        wait_gather(b)
        scale_slot(b)
        start_scatter(b, j0 + b)
    for b in range(NBUF):
        wait_scatter(b)


@jax.jit
def kernel(x, table):
    S0, S1 = x.shape
    B = S0 * S1
    n_chunks = B // (NW * C)
    idx = x.reshape(NW, n_chunks, C).astype(jnp.int32)

    mesh = plsc.VectorSubcoreMesh(core_axis_name="c", subcore_axis_name="s")
    out = pl.kernel(
        _body,
        out_type=jax.ShapeDtypeStruct((B, D_MODEL), jnp.float32),
        mesh=mesh,
        scratch_types=(
            [pltpu.VMEM((n_chunks, C), jnp.int32),
             pltpu.VMEM((NBUF, C, D_MODEL), jnp.float32)]
            + [pltpu.SemaphoreType.DMA] * (2 * NBUF)
        ),
    )(idx, table)
    return out.reshape(S0, S1, D_MODEL)


# speedup vs baseline: 1.8254x; 1.0017x over previous
import functools
import math

import jax
import jax.numpy as jnp
from jax import lax
from jax.experimental import pallas as pl
from jax.experimental.pallas import tpu as pltpu
from jax.experimental.pallas import tpu_sc as plsc

D_MODEL = 128
SCALE = math.sqrt(float(D_MODEL))

NC, NS, L = 2, 16, 16
NW = NC * NS
C = 64
NBUF = 8


def _body(idx_hbm, table_hbm, out_hbm, idx_v, rows_v, *sems):
    gsem = sems[:NBUF]
    ssem = sems[NBUF:]
    n_chunks = idx_v.shape[0]
    n_super = n_chunks // NBUF

    wid = lax.axis_index("s") * NC + lax.axis_index("c")
    base = wid * n_chunks

    pltpu.sync_copy(idx_hbm.at[wid], idx_v)

    def start_gather(b, j):
        pltpu.async_copy(table_hbm.at[idx_v.at[j]], rows_v.at[b], gsem[b])

    def wait_gather(b):
        pltpu.make_async_copy(table_hbm.at[idx_v.at[0]], rows_v.at[b],
                              gsem[b]).wait()

    def start_scatter(b, j):
        pltpu.async_copy(rows_v.at[b],
                         out_hbm.at[pl.ds((base + j) * C, C)], ssem[b])

    def wait_scatter(b):
        pltpu.make_async_copy(rows_v.at[b], out_hbm.at[pl.ds(0, C)],
                              ssem[b]).wait()

    def scale_slot(b):
        def row(r, carry):
            for c8 in range(D_MODEL // L):
                sl = pl.ds(c8 * L, L)
                rows_v[b, r, sl] = rows_v[b, r, sl] * SCALE
            return carry
        lax.fori_loop(0, C, row, None, unroll=4)

    for b in range(NBUF):
        start_gather(b, b)

    def super_body(k, carry):
        j0 = k * NBUF
        for b in range(NBUF):
            wait_gather(b)
            scale_slot(b)
            start_scatter(b, j0 + b)
        for b in range(NBUF):
            wait_scatter(b)
            start_gather(b, j0 + NBUF + b)
        return carry

    lax.fori_loop(0, n_super - 1, super_body, None)

    j0 = (n_super - 1) * NBUF
    for b in range(NBUF):
        wait_gather(b)
        scale_slot(b)
        start_scatter(b, j0 + b)
    for b in range(NBUF):
        wait_scatter(b)


@jax.jit
def kernel(x, table):
    S0, S1 = x.shape
    B = S0 * S1
    n_chunks = B // (NW * C)
    idx = x.reshape(NW, n_chunks, C).astype(jnp.int32)

    mesh = plsc.VectorSubcoreMesh(core_axis_name="c", subcore_axis_name="s")
    out = pl.kernel(
        _body,
        out_type=jax.ShapeDtypeStruct((B, D_MODEL), jnp.float32),
        mesh=mesh,
        scratch_types=(
            [pltpu.VMEM((n_chunks, C), jnp.int32),
             pltpu.VMEM((NBUF, C, D_MODEL), jnp.float32)]
            + [pltpu.SemaphoreType.DMA] * (2 * NBUF)
        ),
    )(idx, table)
    return out.reshape(S0, S1, D_MODEL)
